# baseline (speedup 1.0000x reference)
import functools

import jax
import jax.numpy as jnp
from jax import lax
from jax.experimental import pallas as pl
from jax.experimental.pallas import tpu as pltpu
from jax.experimental.pallas import tpu_sc as plsc

N = 10000
D = 128
E = 320000
NC = 2
NS = 16
NW = NC * NS
CH = 128
EPT = 10240
EPAD = EPT * NW
NCHUNK = EPT // CH
CPH = NCHUNK // 2
NACC = N + 16
TRASH = N + 8
RPT = 624
ZR = 16
NBUF = 2


def _sc_scatter_body(x_hbm, src_hbm, dst_hbm, part_hbm,
                     src_v, dst_v, rows_v, zbuf_v, acc_sh, sem_z, sem_g):
    c = lax.axis_index("c")
    s = lax.axis_index("s")
    wid = c * NS + s

    zv = jnp.zeros((16,), jnp.float32)
    for i in range(ZR):
        for k in range(D // 16):
            zbuf_v[i, pl.ds(k * 16, 16)] = zv

    nz = RPT // ZR
    for j in range(nz):
        pltpu.async_copy(zbuf_v, acc_sh.at[pl.ds(s * RPT + j * ZR, ZR)], sem_z)

    @pl.when(s == NS - 1)
    def _():
        pltpu.async_copy(zbuf_v, acc_sh.at[pl.ds(NS * RPT, ZR)], sem_z)
        pltpu.async_copy(zbuf_v, acc_sh.at[pl.ds(N, ZR)], sem_z)

    for j in range(nz):
        pltpu.make_async_copy(zbuf_v, acc_sh.at[pl.ds(s * RPT + j * ZR, ZR)],
                              sem_z).wait()

    @pl.when(s == NS - 1)
    def _():
        pltpu.make_async_copy(zbuf_v, acc_sh.at[pl.ds(NS * RPT, ZR)],
                              sem_z).wait()
        pltpu.make_async_copy(zbuf_v, acc_sh.at[pl.ds(N, ZR)], sem_z).wait()
    plsc.subcore_barrier()

    for h in range(2):
        pltpu.sync_copy(src_hbm.at[wid, pl.ds(h * CPH, CPH)], src_v)
        pltpu.sync_copy(dst_hbm.at[wid, pl.ds(h * CPH, CPH)], dst_v)

        def chunk_(j, _):
            pltpu.async_copy(x_hbm.at[src_v.at[j]], rows_v.at[0],
                             sem_g.at[0]).wait()
            pltpu.sync_copy(rows_v.at[0], acc_sh.at[dst_v.at[j]], add=True)
            return 0
        lax.fori_loop(0, CPH, chunk_, 0)
    plsc.subcore_barrier()

    pltpu.sync_copy(acc_sh.at[pl.ds(s * RPT, RPT)],
                    part_hbm.at[c, pl.ds(s * RPT, RPT)])

    @pl.when(s == NS - 1)
    def _():
        pltpu.sync_copy(acc_sh.at[pl.ds(NS * RPT, ZR)],
                        part_hbm.at[c, pl.ds(NS * RPT, ZR)])


def _sc_scatter(h, src, dst):
    mesh = plsc.VectorSubcoreMesh(core_axis_name="c", subcore_axis_name="s")
    k = pl.kernel(
        _sc_scatter_body,
        out_type=jax.ShapeDtypeStruct((NC, N, D), jnp.float32),
        mesh=mesh,
        scratch_types=[
            pltpu.VMEM((CPH, CH), jnp.int32),
            pltpu.VMEM((CPH, CH), jnp.int32),
            pltpu.VMEM((NBUF, CH, D), jnp.float32),
            pltpu.VMEM((ZR, D), jnp.float32),
            pltpu.VMEM_SHARED((NACC, D), jnp.float32),
            pltpu.SemaphoreType.DMA,
            pltpu.SemaphoreType.DMA((NBUF,)),
        ],
    )
    return k(h, src, dst)


def _mmbf(a, b):
    return jnp.dot(a.astype(jnp.bfloat16), b.astype(jnp.bfloat16),
                   preferred_element_type=jnp.float32)


def _rbf(a):
    return a.astype(jnp.bfloat16).astype(jnp.float32)


def _tc_layer_body(pp_ref, x_ref, W_ref, b_ref, Wr_ref, br_ref, o_ref):
    agg = pp_ref[0] + pp_ref[1]
    new = jnp.maximum(_mmbf(agg, W_ref[...]) + b_ref[...], 0.0)
    res = jnp.maximum(_mmbf(x_ref[...], Wr_ref[...]) + br_ref[...], 0.0)
    o_ref[...] = new + res


_LB = 1000
_LG = N // _LB


def _tc_layer(part, x, W, b, Wr, br):
    return pl.pallas_call(
        _tc_layer_body,
        grid=(_LG,),
        in_specs=[
            pl.BlockSpec((NC, _LB, D), lambda i: (0, i, 0)),
            pl.BlockSpec((_LB, D), lambda i: (i, 0)),
            pl.BlockSpec((D, D), lambda i: (0, 0)),
            pl.BlockSpec((1, D), lambda i: (0, 0)),
            pl.BlockSpec((D, D), lambda i: (0, 0)),
            pl.BlockSpec((1, D), lambda i: (0, 0)),
        ],
        out_specs=pl.BlockSpec((_LB, D), lambda i: (i, 0)),
        out_shape=jax.ShapeDtypeStruct((N, D), jnp.float32),
    )(part, x, W, b, Wr, br)


def _tc_readout_body(pp_ref, h1_ref, W_ref, b_ref, Wr_ref, br_ref,
                     WgT_ref, bg_ref, Wp1a_ref, Wp1b_ref, bp1_ref,
                     Wp2T_ref, bp2_ref, o_ref, acc_s, acc_m):
    i = pl.program_id(0)
    agg = pp_ref[0] + pp_ref[1]
    new = jnp.maximum(_mmbf(agg, W_ref[...]) + b_ref[...], 0.0)
    res = jnp.maximum(_mmbf(h1_ref[...], Wr_ref[...]) + br_ref[...], 0.0)
    h2 = new + res
    z = (jnp.sum(_rbf(h2) * _rbf(WgT_ref[...]), axis=1, keepdims=True)
         + bg_ref[0, 0])
    wg = 1.0 / (1.0 + jnp.exp(-z))
    s_part = jnp.sum(wg * h2, axis=0, keepdims=True)
    m_part = jnp.max(h2, axis=0, keepdims=True)

    @pl.when(i == 0)
    def _():
        acc_s[...] = s_part
        acc_m[...] = m_part

    @pl.when(i > 0)
    def _():
        acc_s[...] = acc_s[...] + s_part
        acc_m[...] = jnp.maximum(acc_m[...], m_part)

    @pl.when(i == _LG - 1)
    def _():
        g1 = _mmbf(acc_s[...], Wp1a_ref[...])
        g2 = _mmbf(acc_m[...], Wp1b_ref[...])
        h1p = jnp.maximum(g1 + g2 + bp1_ref[...], 0.0)
        h1p = h1p * (1.0 / jnp.sqrt(1.0 + 1e-5))
        o_ref[...] = (jnp.sum(h1p * Wp2T_ref[...], axis=1, keepdims=True)
                      + bp2_ref[...])


def _tc_readout(part, h1, W, b, Wr, br, WgT, bg, Wp1a, Wp1b, bp1, Wp2T, bp2):
    full = lambda shape: pl.BlockSpec(shape, lambda i: tuple(0 for _ in shape))
    return pl.pallas_call(
        _tc_readout_body,
        grid=(_LG,),
        in_specs=[
            pl.BlockSpec((NC, _LB, D), lambda i: (0, i, 0)),
            pl.BlockSpec((_LB, D), lambda i: (i, 0)),
            full((D, D)), full((1, D)), full((D, D)), full((1, D)),
            full((1, D)), full((1, 1)),
            full((D, D)), full((D, D)), full((1, D)),
            full((1, D)), full((1, 1)),
        ],
        out_specs=pl.BlockSpec((1, 1), lambda i: (0, 0)),
        out_shape=jax.ShapeDtypeStruct((1, 1), jnp.float32),
        scratch_shapes=[
            pltpu.VMEM((1, D), jnp.float32),
            pltpu.VMEM((1, D), jnp.float32),
        ],
    )(part, h1, W, b, Wr, br, WgT, bg, Wp1a, Wp1b, bp1, Wp2T, bp2)


def kernel(x, edge_index, W1, b1, Wr1, br1, W2, b2, Wr2, br2,
           Wg, bg, Wp1, bp1, Wp2, bp2):
    pad = EPAD - E
    idt = edge_index.dtype
    src = jnp.concatenate(
        [edge_index[0], jnp.zeros((pad,), idt)]).reshape(NW, NCHUNK, CH)
    dst = jnp.concatenate(
        [edge_index[1], jnp.full((pad,), TRASH, idt)]).reshape(NW, NCHUNK, CH)

    part1 = _sc_scatter(x, src, dst)
    h1 = _tc_layer(part1, x, W1, b1.reshape(1, D), Wr1, br1.reshape(1, D))
    part2 = _sc_scatter(h1, src, dst)
    out = _tc_readout(
        part2, h1, W2, b2.reshape(1, D), Wr2, br2.reshape(1, D),
        Wg.reshape(1, D), bg.reshape(1, 1),
        Wp1[:D], Wp1[D:], bp1.reshape(1, D),
        Wp2.reshape(1, D), bp2.reshape(1, 1),
    )
    return out


# speedup vs baseline: 3.5862x; 3.5862x over previous
import functools

import jax
import jax.numpy as jnp
from jax import lax
from jax.experimental import pallas as pl
from jax.experimental.pallas import tpu as pltpu
from jax.experimental.pallas import tpu_sc as plsc

N = 10000
D = 128
E = 320000
NC = 2
NS = 16
NW = NC * NS
CH = 80
EPT = E // NW
NCHUNK = EPT // CH
NSTAGE = 5
CPS = NCHUNK // NSTAGE
RPT = 624
ZR = 16


def _sc_scatter_body(x_hbm, src_hbm, dst_hbm, part_hbm,
                     src_v, dst_v, rows0_v, rows1_v, zbuf_v, acc_sh,
                     sem_z, sem0, sem1):
    c = lax.axis_index("c")
    s = lax.axis_index("s")
    wid = c * NS + s

    zv = jnp.zeros((16,), jnp.float32)
    for i in range(ZR):
        for k in range(D // 16):
            zbuf_v[i, pl.ds(k * 16, 16)] = zv

    nz = RPT // ZR
    for j in range(nz):
        pltpu.async_copy(zbuf_v, acc_sh.at[pl.ds(s * RPT + j * ZR, ZR)], sem_z)

    @pl.when(s == NS - 1)
    def _():
        pltpu.async_copy(zbuf_v, acc_sh.at[pl.ds(NS * RPT, ZR)], sem_z)

    for j in range(nz):
        pltpu.make_async_copy(zbuf_v, acc_sh.at[pl.ds(s * RPT + j * ZR, ZR)],
                              sem_z).wait()

    @pl.when(s == NS - 1)
    def _():
        pltpu.make_async_copy(zbuf_v, acc_sh.at[pl.ds(NS * RPT, ZR)],
                              sem_z).wait()
    plsc.subcore_barrier()

    for st in range(NSTAGE):
        pltpu.sync_copy(src_hbm.at[wid, st], src_v)
        pltpu.sync_copy(dst_hbm.at[wid, st], dst_v)

        pltpu.async_copy(x_hbm.at[src_v.at[0]], rows0_v, sem0)
        pltpu.async_copy(x_hbm.at[src_v.at[1]], rows1_v, sem1)

        def round_(t, _):
            c0 = 2 * t
            pltpu.make_async_copy(x_hbm.at[src_v.at[c0]], rows0_v,
                                  sem0).wait()
            pltpu.sync_copy(rows0_v, acc_sh.at[dst_v.at[c0]], add=True)
            pltpu.async_copy(x_hbm.at[src_v.at[c0 + 2]], rows0_v, sem0)
            pltpu.make_async_copy(x_hbm.at[src_v.at[c0 + 1]], rows1_v,
                                  sem1).wait()
            pltpu.sync_copy(rows1_v, acc_sh.at[dst_v.at[c0 + 1]], add=True)

            @pl.when(t < CPS // 2 - 1)
            def _():
                pltpu.async_copy(x_hbm.at[src_v.at[c0 + 3]], rows1_v, sem1)
            return 0
        lax.fori_loop(0, CPS // 2, round_, 0)

        pltpu.make_async_copy(x_hbm.at[src_v.at[CPS - 1]], rows0_v,
                              sem0).wait()
        pltpu.sync_copy(rows0_v, acc_sh.at[dst_v.at[CPS - 1]], add=True)
    plsc.subcore_barrier()

    pltpu.sync_copy(acc_sh.at[pl.ds(s * RPT, RPT)],
                    part_hbm.at[c, pl.ds(s * RPT, RPT)])

    @pl.when(s == NS - 1)
    def _():
        pltpu.sync_copy(acc_sh.at[pl.ds(NS * RPT, ZR)],
                        part_hbm.at[c, pl.ds(NS * RPT, ZR)])


def _sc_scatter(h, src, dst):
    mesh = plsc.VectorSubcoreMesh(core_axis_name="c", subcore_axis_name="s")
    k = pl.kernel(
        _sc_scatter_body,
        out_type=jax.ShapeDtypeStruct((NC, N, D), jnp.float32),
        mesh=mesh,
        scratch_types=[
            pltpu.VMEM((CPS, CH), jnp.int32),
            pltpu.VMEM((CPS, CH), jnp.int32),
            pltpu.VMEM((CH, D), jnp.float32),
            pltpu.VMEM((CH, D), jnp.float32),
            pltpu.VMEM((ZR, D), jnp.float32),
            pltpu.VMEM_SHARED((N, D), jnp.float32),
            pltpu.SemaphoreType.DMA,
            pltpu.SemaphoreType.DMA,
            pltpu.SemaphoreType.DMA,
        ],
    )
    return k(h, src, dst)


def _mmbf(a, b):
    return jnp.dot(a.astype(jnp.bfloat16), b.astype(jnp.bfloat16),
                   preferred_element_type=jnp.float32)


def _rbf(a):
    return a.astype(jnp.bfloat16).astype(jnp.float32)


def _tc_layer_body(pp_ref, x_ref, W_ref, b_ref, Wr_ref, br_ref, o_ref):
    agg = pp_ref[0] + pp_ref[1]
    new = jnp.maximum(_mmbf(agg, W_ref[...]) + b_ref[...], 0.0)
    res = jnp.maximum(_mmbf(x_ref[...], Wr_ref[...]) + br_ref[...], 0.0)
    o_ref[...] = new + res


_LB = 1000
_LG = N // _LB


def _tc_layer(part, x, W, b, Wr, br):
    return pl.pallas_call(
        _tc_layer_body,
        grid=(_LG,),
        in_specs=[
            pl.BlockSpec((NC, _LB, D), lambda i: (0, i, 0)),
            pl.BlockSpec((_LB, D), lambda i: (i, 0)),
            pl.BlockSpec((D, D), lambda i: (0, 0)),
            pl.BlockSpec((1, D), lambda i: (0, 0)),
            pl.BlockSpec((D, D), lambda i: (0, 0)),
            pl.BlockSpec((1, D), lambda i: (0, 0)),
        ],
        out_specs=pl.BlockSpec((_LB, D), lambda i: (i, 0)),
        out_shape=jax.ShapeDtypeStruct((N, D), jnp.float32),
    )(part, x, W, b, Wr, br)


def _tc_readout_body(pp_ref, h1_ref, W_ref, b_ref, Wr_ref, br_ref,
                     WgT_ref, bg_ref, Wp1a_ref, Wp1b_ref, bp1_ref,
                     Wp2T_ref, bp2_ref, o_ref, acc_s, acc_m):
    i = pl.program_id(0)
    agg = pp_ref[0] + pp_ref[1]
    new = jnp.maximum(_mmbf(agg, W_ref[...]) + b_ref[...], 0.0)
    res = jnp.maximum(_mmbf(h1_ref[...], Wr_ref[...]) + br_ref[...], 0.0)
    h2 = new + res
    z = (jnp.sum(_rbf(h2) * _rbf(WgT_ref[...]), axis=1, keepdims=True)
         + bg_ref[0, 0])
    wg = 1.0 / (1.0 + jnp.exp(-z))
    s_part = jnp.sum(wg * h2, axis=0, keepdims=True)
    m_part = jnp.max(h2, axis=0, keepdims=True)

    @pl.when(i == 0)
    def _():
        acc_s[...] = s_part
        acc_m[...] = m_part

    @pl.when(i > 0)
    def _():
        acc_s[...] = acc_s[...] + s_part
        acc_m[...] = jnp.maximum(acc_m[...], m_part)

    @pl.when(i == _LG - 1)
    def _():
        g1 = _mmbf(acc_s[...], Wp1a_ref[...])
        g2 = _mmbf(acc_m[...], Wp1b_ref[...])
        h1p = jnp.maximum(g1 + g2 + bp1_ref[...], 0.0)
        h1p = h1p * (1.0 / jnp.sqrt(1.0 + 1e-5))
        o_ref[...] = (jnp.sum(h1p * Wp2T_ref[...], axis=1, keepdims=True)
                      + bp2_ref[...])


def _tc_readout(part, h1, W, b, Wr, br, WgT, bg, Wp1a, Wp1b, bp1, Wp2T, bp2):
    full = lambda shape: pl.BlockSpec(shape, lambda i: tuple(0 for _ in shape))
    return pl.pallas_call(
        _tc_readout_body,
        grid=(_LG,),
        in_specs=[
            pl.BlockSpec((NC, _LB, D), lambda i: (0, i, 0)),
            pl.BlockSpec((_LB, D), lambda i: (i, 0)),
            full((D, D)), full((1, D)), full((D, D)), full((1, D)),
            full((1, D)), full((1, 1)),
            full((D, D)), full((D, D)), full((1, D)),
            full((1, D)), full((1, 1)),
        ],
        out_specs=pl.BlockSpec((1, 1), lambda i: (0, 0)),
        out_shape=jax.ShapeDtypeStruct((1, 1), jnp.float32),
        scratch_shapes=[
            pltpu.VMEM((1, D), jnp.float32),
            pltpu.VMEM((1, D), jnp.float32),
        ],
    )(part, h1, W, b, Wr, br, WgT, bg, Wp1a, Wp1b, bp1, Wp2T, bp2)


def kernel(x, edge_index, W1, b1, Wr1, br1, W2, b2, Wr2, br2,
           Wg, bg, Wp1, bp1, Wp2, bp2):
    src = edge_index[0].reshape(NW, NSTAGE, CPS, CH)
    dst = edge_index[1].reshape(NW, NSTAGE, CPS, CH)

    part1 = _sc_scatter(x, src, dst)
    h1 = _tc_layer(part1, x, W1, b1.reshape(1, D), Wr1, br1.reshape(1, D))
    part2 = _sc_scatter(h1, src, dst)
    out = _tc_readout(
        part2, h1, W2, b2.reshape(1, D), Wr2, br2.reshape(1, D),
        Wg.reshape(1, D), bg.reshape(1, 1),
        Wp1[:D], Wp1[D:], bp1.reshape(1, D),
        Wp2.reshape(1, D), bp2.reshape(1, 1),
    )
    return out
